# baseline (speedup 1.0000x reference)
import math
import functools

import jax
import jax.numpy as jnp
import numpy as np
from jax import lax
from jax.experimental import pallas as pl
from jax.experimental.pallas import tpu as pltpu
from jax.experimental.pallas import tpu_sc as plsc

VOCAB = 1000000
D_MODEL = 128
BATCH = 4
SEQ_LEN = 4096

NUM_CORES = 2
NUM_SUBCORES = 16
LANES = 16
NW = NUM_CORES * NUM_SUBCORES
S_PER_W = SEQ_LEN // NW

_SCALE = math.sqrt(D_MODEL)


def _pe_table(seq_len, d_model):
    position = np.arange(seq_len, dtype=np.float32)[:, None]
    div_term = np.exp(
        np.arange(0, d_model, 2, dtype=np.float32) * -(math.log(10000.0) / d_model)
    )
    pe = np.zeros((seq_len, d_model), dtype=np.float32)
    pe[:, 0::2] = np.sin(position * div_term)
    pe[:, 1::2] = np.cos(position * div_term)
    return pe


_PE = _pe_table(SEQ_LEN, D_MODEL)


def _sc_embed(x_flat, pe, table):
    mesh = plsc.VectorSubcoreMesh(core_axis_name="c", subcore_axis_name="s")

    @functools.partial(
        pl.kernel,
        mesh=mesh,
        out_type=jax.ShapeDtypeStruct((BATCH * SEQ_LEN, D_MODEL), jnp.float32),
        scratch_types=[
            pltpu.VMEM((S_PER_W,), jnp.int32),
            pltpu.VMEM((S_PER_W, D_MODEL), jnp.float32),
            pltpu.VMEM((S_PER_W, D_MODEL), jnp.float32),
            pltpu.SemaphoreType.DMA,
        ],
    )
    def k(x_hbm, pe_hbm, table_hbm, out_hbm, idx_v, rows_v, pe_v, sem):
        wid = lax.axis_index("s") * NUM_CORES + lax.axis_index("c")
        pos0 = wid * S_PER_W
        pltpu.sync_copy(pe_hbm.at[pl.ds(pos0, S_PER_W)], pe_v)
        for b in range(BATCH):
            base = b * SEQ_LEN + pos0
            pltpu.sync_copy(x_hbm.at[pl.ds(base, S_PER_W)], idx_v)
            pltpu.async_copy(table_hbm.at[idx_v], rows_v, sem).wait()

            def body(r, _):
                for d in range(D_MODEL // LANES):
                    sl = pl.ds(d * LANES, LANES)
                    v = rows_v[r, sl]
                    p = pe_v[r, sl]
                    rows_v[r, sl] = v * _SCALE + p
                return 0

            lax.fori_loop(0, S_PER_W, body, 0)
            pltpu.sync_copy(rows_v, out_hbm.at[pl.ds(base, S_PER_W)])

    return k(x_flat, pe, table)


@jax.jit
def kernel(x, table):
    x_flat = x.reshape(-1).astype(jnp.int32)
    out = _sc_embed(x_flat, _PE, table)
    return out.reshape(BATCH, SEQ_LEN, D_MODEL)


# speedup vs baseline: 1.1779x; 1.1779x over previous
import math
import functools

import jax
import jax.numpy as jnp
import numpy as np
from jax import lax
from jax.experimental import pallas as pl
from jax.experimental.pallas import tpu as pltpu
from jax.experimental.pallas import tpu_sc as plsc

VOCAB = 1000000
D_MODEL = 128
BATCH = 4
SEQ_LEN = 4096

NUM_CORES = 2
NUM_SUBCORES = 16
LANES = 16
NW = NUM_CORES * NUM_SUBCORES
S_PER_W = SEQ_LEN // NW

_SCALE = math.sqrt(D_MODEL)


def _pe_table(seq_len, d_model):
    position = np.arange(seq_len, dtype=np.float32)[:, None]
    div_term = np.exp(
        np.arange(0, d_model, 2, dtype=np.float32) * -(math.log(10000.0) / d_model)
    )
    pe = np.zeros((seq_len, d_model), dtype=np.float32)
    pe[:, 0::2] = np.sin(position * div_term)
    pe[:, 1::2] = np.cos(position * div_term)
    return pe


_PE = _pe_table(SEQ_LEN, D_MODEL)


def _sc_embed(x_flat, pe, table):
    mesh = plsc.VectorSubcoreMesh(core_axis_name="c", subcore_axis_name="s")

    @functools.partial(
        pl.kernel,
        mesh=mesh,
        out_type=jax.ShapeDtypeStruct((BATCH * SEQ_LEN, D_MODEL), jnp.float32),
        scratch_types=[
            pltpu.VMEM((BATCH, S_PER_W), jnp.int32),
            pltpu.VMEM((S_PER_W, D_MODEL), jnp.float32),
        ]
        + [pltpu.VMEM((S_PER_W, D_MODEL), jnp.float32) for _ in range(BATCH)]
        + [pltpu.SemaphoreType.DMA for _ in range(BATCH)]
        + [pltpu.SemaphoreType.DMA, pltpu.SemaphoreType.DMA],
    )
    def k(x_hbm, pe_hbm, table_hbm, out_hbm, idx_v, pe_v,
          buf0, buf1, buf2, buf3, g0, g1, g2, g3, psem, ssem):
        bufs = [buf0, buf1, buf2, buf3]
        gsems = [g0, g1, g2, g3]
        wid = lax.axis_index("s") * NUM_CORES + lax.axis_index("c")
        pos0 = wid * S_PER_W

        pe_cp = pltpu.async_copy(pe_hbm.at[pl.ds(pos0, S_PER_W)], pe_v, psem)
        for b in range(BATCH):
            pltpu.sync_copy(x_hbm.at[pl.ds(b * SEQ_LEN + pos0, S_PER_W)],
                            idx_v.at[b])
        gathers = [
            pltpu.async_copy(table_hbm.at[idx_v.at[b]], bufs[b], gsems[b])
            for b in range(BATCH)
        ]
        pe_cp.wait()

        stores = []
        for b in range(BATCH):
            buf = bufs[b]
            gathers[b].wait()

            def body(r, _, buf=buf):
                for d in range(D_MODEL // LANES):
                    sl = pl.ds(d * LANES, LANES)
                    buf[r, sl] = buf[r, sl] * _SCALE + pe_v[r, sl]
                return 0

            lax.fori_loop(0, S_PER_W, body, 0)
            stores.append(
                pltpu.async_copy(
                    buf, out_hbm.at[pl.ds(b * SEQ_LEN + pos0, S_PER_W)], ssem
                )
            )
        for st in stores:
            st.wait()

    return k(x_flat, pe, table)


@jax.jit
def kernel(x, table):
    x_flat = x.reshape(-1).astype(jnp.int32)
    out = _sc_embed(x_flat, _PE, table)
    return out.reshape(BATCH, SEQ_LEN, D_MODEL)


# speedup vs baseline: 1.2853x; 1.0912x over previous
import math
import functools

import jax
import jax.numpy as jnp
import numpy as np
from jax import lax
from jax.experimental import pallas as pl
from jax.experimental.pallas import tpu as pltpu
from jax.experimental.pallas import tpu_sc as plsc

VOCAB = 1000000
D_MODEL = 128
BATCH = 4
SEQ_LEN = 4096

NUM_CORES = 2
NUM_SUBCORES = 16
LANES = 16
NW = NUM_CORES * NUM_SUBCORES
S_PER_W = SEQ_LEN // NW

_SCALE = math.sqrt(D_MODEL)


def _pe_table(seq_len, d_model):
    position = np.arange(seq_len, dtype=np.float32)[:, None]
    div_term = np.exp(
        np.arange(0, d_model, 2, dtype=np.float32) * -(math.log(10000.0) / d_model)
    )
    pe = np.zeros((seq_len, d_model), dtype=np.float32)
    pe[:, 0::2] = np.sin(position * div_term)
    pe[:, 1::2] = np.cos(position * div_term)
    return pe


_PE = _pe_table(SEQ_LEN, D_MODEL)


def _sc_embed(x, pe, table):
    mesh = plsc.VectorSubcoreMesh(core_axis_name="c", subcore_axis_name="s")

    @functools.partial(
        pl.kernel,
        mesh=mesh,
        out_type=jax.ShapeDtypeStruct((BATCH, SEQ_LEN, D_MODEL), jnp.float32),
        scratch_types=[
            pltpu.VMEM((BATCH, S_PER_W), jnp.int32),
            pltpu.VMEM((S_PER_W, D_MODEL), jnp.float32),
        ]
        + [pltpu.VMEM((S_PER_W, D_MODEL), jnp.float32) for _ in range(BATCH)]
        + [pltpu.SemaphoreType.DMA for _ in range(BATCH)]
        + [pltpu.SemaphoreType.DMA, pltpu.SemaphoreType.DMA],
    )
    def k(x_hbm, pe_hbm, table_hbm, out_hbm, idx_v, pe_v,
          buf0, buf1, buf2, buf3, g0, g1, g2, g3, psem, ssem):
        bufs = [buf0, buf1, buf2, buf3]
        gsems = [g0, g1, g2, g3]
        wid = lax.axis_index("s") * NUM_CORES + lax.axis_index("c")
        pos0 = wid * S_PER_W

        pe_cp = pltpu.async_copy(pe_hbm.at[pl.ds(pos0, S_PER_W)], pe_v, psem)
        pltpu.sync_copy(x_hbm.at[:, pl.ds(pos0, S_PER_W)], idx_v)
        gathers = [
            pltpu.async_copy(table_hbm.at[idx_v.at[b]], bufs[b], gsems[b])
            for b in range(BATCH)
        ]
        pe_cp.wait()

        stores = []
        for b in range(BATCH):
            buf = bufs[b]
            gathers[b].wait()

            def body(r, _, buf=buf):
                for d in range(D_MODEL // LANES):
                    sl = pl.ds(d * LANES, LANES)
                    buf[r, sl] = buf[r, sl] * _SCALE + pe_v[r, sl]
                return 0

            lax.fori_loop(0, S_PER_W, body, 0)
            stores.append(
                pltpu.async_copy(
                    buf, out_hbm.at[b, pl.ds(pos0, S_PER_W)], ssem
                )
            )
        for st in stores:
            st.wait()

    return k(x, pe, table)


@jax.jit
def kernel(x, table):
    return _sc_embed(x.astype(jnp.int32), _PE, table)


# speedup vs baseline: 1.2972x; 1.0093x over previous
import math
import functools

import jax
import jax.numpy as jnp
import numpy as np
from jax import lax
from jax.experimental import pallas as pl
from jax.experimental.pallas import tpu as pltpu
from jax.experimental.pallas import tpu_sc as plsc

VOCAB = 1000000
D_MODEL = 128
BATCH = 4
SEQ_LEN = 4096

NUM_CORES = 2
NUM_SUBCORES = 16
LANES = 16
NW = NUM_CORES * NUM_SUBCORES
S_PER_W = SEQ_LEN // NW
HALF = S_PER_W // 2

_SCALE = math.sqrt(D_MODEL)


def _pe_table(seq_len, d_model):
    position = np.arange(seq_len, dtype=np.float32)[:, None]
    div_term = np.exp(
        np.arange(0, d_model, 2, dtype=np.float32) * -(math.log(10000.0) / d_model)
    )
    pe = np.zeros((seq_len, d_model), dtype=np.float32)
    pe[:, 0::2] = np.sin(position * div_term)
    pe[:, 1::2] = np.cos(position * div_term)
    return pe


_PE = _pe_table(SEQ_LEN, D_MODEL)
_PE_DEV = None


def _sc_embed(x, pe, table):
    mesh = plsc.VectorSubcoreMesh(core_axis_name="c", subcore_axis_name="s")

    @functools.partial(
        pl.kernel,
        mesh=mesh,
        out_type=jax.ShapeDtypeStruct((BATCH, SEQ_LEN, D_MODEL), jnp.float32),
        scratch_types=[
            pltpu.VMEM((BATCH, S_PER_W), jnp.int32),
            pltpu.VMEM((S_PER_W, D_MODEL), jnp.float32),
        ]
        + [pltpu.VMEM((S_PER_W, D_MODEL), jnp.float32) for _ in range(BATCH)]
        + [pltpu.SemaphoreType.DMA for _ in range(2 * BATCH)]
        + [pltpu.SemaphoreType.DMA, pltpu.SemaphoreType.DMA],
    )
    def k(x_hbm, pe_hbm, table_hbm, out_hbm, idx_v, pe_v,
          buf0, buf1, buf2, buf3,
          g0, g1, g2, g3, g4, g5, g6, g7, psem, ssem):
        bufs = [buf0, buf1, buf2, buf3]
        gsems = [g0, g1, g2, g3, g4, g5, g6, g7]
        wid = lax.axis_index("s") * NUM_CORES + lax.axis_index("c")
        pos0 = wid * S_PER_W

        pe_cp = pltpu.async_copy(pe_hbm.at[pl.ds(pos0, S_PER_W)], pe_v, psem)
        pltpu.sync_copy(x_hbm.at[:, pl.ds(pos0, S_PER_W)], idx_v)
        gathers = []
        for b in range(BATCH):
            for h in range(2):
                gathers.append(pltpu.async_copy(
                    table_hbm.at[idx_v.at[b, pl.ds(h * HALF, HALF)]],
                    bufs[b].at[pl.ds(h * HALF, HALF)],
                    gsems[2 * b + h],
                ))
        pe_cp.wait()

        stores = []
        for b in range(BATCH):
            buf = bufs[b]
            for h in range(2):
                gathers[2 * b + h].wait()
                r0 = h * HALF

                def body(r, _, buf=buf, r0=r0):
                    for d in range(D_MODEL // LANES):
                        sl = pl.ds(d * LANES, LANES)
                        buf[r0 + r, sl] = buf[r0 + r, sl] * _SCALE + pe_v[r0 + r, sl]
                    return 0

                lax.fori_loop(0, HALF, body, 0)
                stores.append(
                    pltpu.async_copy(
                        buf.at[pl.ds(r0, HALF)],
                        out_hbm.at[b, pl.ds(pos0 + r0, HALF)],
                        ssem,
                    )
                )
        for st in stores:
            st.wait()

    return k(x, pe, table)


@functools.partial(jax.jit)
def _run(x, pe, table):
    return _sc_embed(x.astype(jnp.int32), pe, table)


def kernel(x, table):
    global _PE_DEV
    if _PE_DEV is None:
        _PE_DEV = jnp.asarray(_PE)
    return _run(x, _PE_DEV, table)
